# baseline (speedup 1.0000x reference)
import functools

import jax
import jax.numpy as jnp
from jax import lax
from jax.experimental import pallas as pl
from jax.experimental.pallas import tpu as pltpu
from jax.experimental.pallas import tpu_sc as plsc

_B = 1024
_T = 50
_BB = 16
_BBT = _BB * _T
_GRID = _B // _BB

_D_IMG = 2048
_D_EMB = 64
_D_CAT = 23


_NC = 2
_NS = 16
_NW = _NC * _NS
_HPW = (_B * _T) // _NW
_QPW = _B // _NW
_CK = 80
_NCK = _HPW // _CK


def _sc_gather_body(ht, hi, qt, qi, h_out, q_out,
                    hidx_v, hrows_v, qidx_v, qrows_v, sem):
    c = lax.axis_index("c")
    s = lax.axis_index("s")
    wid = s * _NC + c
    hb = wid * _HPW
    qb = wid * _QPW
    pltpu.sync_copy(hi.at[pl.ds(hb, _HPW)], hidx_v)
    pltpu.sync_copy(qi.at[pl.ds(qb, _QPW)], qidx_v)
    cps = []
    for ck in range(_NCK):
        cps.append(pltpu.async_copy(
            ht.at[hidx_v.at[pl.ds(ck * _CK, _CK)]],
            hrows_v.at[pl.ds(ck * _CK, _CK)], sem))
    cps.append(pltpu.async_copy(qt.at[qidx_v], qrows_v, sem))
    for cp in cps:
        cp.wait()
    pltpu.sync_copy(hrows_v, h_out.at[pl.ds(hb, _HPW)])
    pltpu.sync_copy(qrows_v, q_out.at[pl.ds(qb, _QPW)])


@functools.lru_cache(maxsize=None)
def _sc_gather_fn():
    return functools.partial(
        pl.kernel,
        mesh=plsc.VectorSubcoreMesh(core_axis_name="c", subcore_axis_name="s",
                                    num_cores=_NC, num_subcores=_NS),
        out_type=[
            jax.ShapeDtypeStruct((_B * _T, _D_EMB), jnp.float32),
            jax.ShapeDtypeStruct((_B, _D_EMB), jnp.float32),
        ],
        scratch_types=[
            pltpu.VMEM((_HPW,), jnp.int32),
            pltpu.VMEM((_HPW, _D_EMB), jnp.float32),
            pltpu.VMEM((_QPW,), jnp.int32),
            pltpu.VMEM((_QPW, _D_EMB), jnp.float32),
            pltpu.SemaphoreType.DMA,
        ],
        compiler_params=pltpu.CompilerParams(use_tc_tiling_on_sc=False),
    )(_sc_gather_body)


def _din_block(hif, hemb, hcat, qemb, qimg, qcat, uet, ug, ua, hlen,
               Wqi, bqi, Whi, bhi,
               Wq_e, Wq_i, Wq_c, Wh_e, Wh_i, Wh_c, Wp_e, Wp_i, Wp_c, ba1,
               Wa2, ba2, Wa3, ba3,
               W1u1, W1u2, W1u3, W1qe, W1qi, W1qc, W1pe, W1pi, W1pc, b1,
               W2, b2, W3, out):
    f32 = jnp.float32
    dot = functools.partial(jnp.dot, preferred_element_type=f32)

    q_img = jax.nn.relu(dot(qimg[...], Wqi[...]) + bqi[...])
    q_emb = qemb[...]
    q_cat = qcat[...]

    h_img = jax.nn.relu(dot(hif[...], Whi[...]) + bhi[...])
    h_emb = hemb[...]
    h_cat = hcat[...]

    ri = lax.broadcasted_iota(jnp.int32, (_BBT, _BB), 0)
    ci = lax.broadcasted_iota(jnp.int32, (_BBT, _BB), 1)
    S = (ri // _T == ci).astype(f32)
    rj = lax.broadcasted_iota(jnp.int32, (_BB, _BBT), 1)
    cj = lax.broadcasted_iota(jnp.int32, (_BB, _BBT), 0)
    St = (rj // _T == cj).astype(f32)

    qc = (dot(q_emb, Wq_e[...]) + dot(q_img, Wq_i[...])
          + dot(q_cat, Wq_c[...]) + ba1[...])
    q_emb_b = dot(S, q_emb)
    q_img_b = dot(S, q_img)
    q_cat_b = dot(S, q_cat)

    a1 = jax.nn.relu(
        dot(S, qc)
        + dot(h_emb, Wh_e[...]) + dot(h_img, Wh_i[...]) + dot(h_cat, Wh_c[...])
        + dot(q_emb_b * h_emb, Wp_e[...])
        + dot(q_img_b * h_img, Wp_i[...])
        + dot(q_cat_b * h_cat, Wp_c[...]))
    a2 = jax.nn.relu(dot(a1, Wa2[...]) + ba2[...])
    s = dot(a2, Wa3[...]) + ba3[...]

    tpos = (lax.broadcasted_iota(jnp.int32, (_BBT, 1), 0) % _T).astype(f32)
    len_b = dot(S, hlen[...])
    s = jnp.where(tpos < len_b, s, f32(-1e9))
    e = jnp.exp(s - jnp.max(s))
    denom = dot(St, e)
    inv = 1.0 / denom
    p_e = dot(St, e * h_emb) * inv
    p_i = dot(St, e * h_img) * inv
    p_c = dot(St, e * h_cat) * inv

    o1 = jax.nn.relu(
        dot(uet[...], W1u1[...]) + dot(ug[...], W1u2[...]) + dot(ua[...], W1u3[...])
        + dot(q_emb, W1qe[...]) + dot(q_img, W1qi[...]) + dot(q_cat, W1qc[...])
        + dot(p_e, W1pe[...]) + dot(p_i, W1pi[...]) + dot(p_c, W1pc[...])
        + b1[...])
    o2 = jax.nn.relu(dot(o1, W2[...]) + b2[...])
    out[...] = jax.nn.sigmoid(dot(o2, W3[...]))


def _blk(shape, row_block):
    return pl.BlockSpec((row_block,) + shape[1:], lambda i: (i,) + (0,) * (len(shape) - 1))


def _const(shape):
    return pl.BlockSpec(shape, lambda i: (0,) * len(shape))


_TC_IN_SPECS = [
    _blk((_B * _T, _D_IMG), _BBT),
    _blk((_B * _T, _D_EMB), _BBT),
    _blk((_B * _T, _D_CAT), _BBT),
    _blk((_B, _D_EMB), _BB),
    _blk((_B, _D_IMG), _BB),
    _blk((_B, _D_CAT), _BB),
    _blk((_B, 24), _BB),
    _blk((_B, 2), _BB),
    _blk((_B, 9), _BB),
    _blk((_B, 1), _BB),
    _const((_D_IMG, 64)), _const((1, 64)),
    _const((_D_IMG, 64)), _const((1, 64)),
    _const((64, 80)), _const((64, 80)), _const((23, 80)),
    _const((64, 80)), _const((64, 80)), _const((23, 80)),
    _const((64, 80)), _const((64, 80)), _const((23, 80)),
    _const((1, 80)),
    _const((80, 40)), _const((1, 40)),
    _const((40, 1)), _const((1, 1)),
    _const((24, 200)), _const((2, 200)), _const((9, 200)),
    _const((64, 200)), _const((64, 200)), _const((23, 200)),
    _const((64, 200)), _const((64, 200)), _const((23, 200)),
    _const((1, 200)),
    _const((200, 80)), _const((1, 80)),
    _const((80, 1)),
]

_TC_OUT_SPEC = pl.BlockSpec((_BB, 1), lambda i: (i, 0))


def _fused_tc(*args, interpret=False):
    return pl.pallas_call(
        _din_block,
        grid=(_GRID,),
        in_specs=_TC_IN_SPECS,
        out_specs=_TC_OUT_SPEC,
        out_shape=jax.ShapeDtypeStruct((_B, 1), jnp.float32),
        interpret=interpret,
    )(*args)


def kernel(user_exposed_time, user_gender, user_age, query_article_id,
           query_image_feature, query_categories, history_article_id,
           history_image_feature, history_categories, history_len,
           query_emb_table, hist_emb_table, Wqi, bqi, Whi, bhi,
           Wa1, ba1, Wa2, ba2, Wa3, ba3, W1, b1, W2, b2, W3):
    f32 = jnp.float32
    D = 151

    hidx = history_article_id.reshape(_B * _T).astype(jnp.int32)
    qidx = query_article_id.astype(jnp.int32)
    hemb, qemb = _sc_gather_fn()(hist_emb_table, hidx, query_emb_table, qidx)

    hif2 = history_image_feature.reshape(_B * _T, _D_IMG)
    hcat2 = history_categories.reshape(_B * _T, _D_CAT)
    hlen = history_len.astype(f32).reshape(_B, 1)

    Aq, Ah, Ad, Ap = Wa1[0:D], Wa1[D:2 * D], Wa1[2 * D:3 * D], Wa1[3 * D:4 * D]
    Wq = Aq + Ad
    Wh = Ah - Ad
    Wq_e, Wq_i, Wq_c = Wq[0:64], Wq[64:128], Wq[128:151]
    Wh_e, Wh_i, Wh_c = Wh[0:64], Wh[64:128], Wh[128:151]
    Wp_e, Wp_i, Wp_c = Ap[0:64], Ap[64:128], Ap[128:151]

    W1u1, W1u2, W1u3 = W1[0:24], W1[24:26], W1[26:35]
    W1qe, W1qi, W1qc = W1[35:99], W1[99:163], W1[163:186]
    W1pe, W1pi, W1pc = W1[186:250], W1[250:314], W1[314:337]

    return _fused_tc(
        hif2, hemb, hcat2, qemb, query_image_feature, query_categories,
        user_exposed_time, user_gender, user_age, hlen,
        Wqi, bqi.reshape(1, 64), Whi, bhi.reshape(1, 64),
        Wq_e, Wq_i, Wq_c, Wh_e, Wh_i, Wh_c, Wp_e, Wp_i, Wp_c,
        ba1.reshape(1, 80), Wa2, ba2.reshape(1, 40), Wa3, ba3.reshape(1, 1),
        W1u1, W1u2, W1u3, W1qe, W1qi, W1qc, W1pe, W1pi, W1pc,
        b1.reshape(1, 200), W2, b2.reshape(1, 80), W3)
